# baseline (speedup 1.0000x reference)
import functools

import jax
import jax.numpy as jnp
from jax import lax
from jax.experimental import pallas as pl
from jax.experimental.pallas import tpu as pltpu
from jax.experimental.pallas import tpu_sc as plsc

_TOPK = 5
_RADIUS = 5
_ROWS = 64
_N = 32768
_L = 16
_C = 256
_NCHUNK = _N // _C
_NVPC = _C // _L
_NSUP = 8
_CPS = _NCHUNK // _NSUP
_SEC = _N // _NSUP
_BIG = 1 << 30

_mesh = plsc.VectorSubcoreMesh(
    core_axis_name="c", subcore_axis_name="s", num_cores=1, num_subcores=16
)
_RPT = 4


def _rescan_chunk(row_v, cmax_v, c):
    base = c * _C
    acc = row_v[pl.ds(base, _L)]
    for i in range(1, _NVPC):
        acc = jnp.maximum(acc, row_v[pl.ds(base + i * _L, _L)])
    cmax_v[pl.ds(c * _L, _L)] = acc


def _rebuild_super(cmax_v, smax_v, s):
    base = s * _CPS * _L
    acc = cmax_v[pl.ds(base, _L)]
    for k in range(1, _CPS):
        acc = jnp.maximum(acc, cmax_v[pl.ds(base + k * _L, _L)])
    smax_v[pl.ds(s * _L, _L)] = acc


def _compute_picks(row_v, cmax_v, smax_v, iota, sec_dmas):
    sec_dmas.wait()
    gminv = jnp.full((_L,), jnp.inf, jnp.float32)
    for s in range(_NSUP):

        def _cbody(i, carry):
            sacc, gmin = carry
            base = (s * _CPS + i) * _C
            acc = row_v[pl.ds(base, _L)]
            accmin = acc
            for k in range(1, _NVPC):
                v = row_v[pl.ds(base + k * _L, _L)]
                acc = jnp.maximum(acc, v)
                accmin = jnp.minimum(accmin, v)
            cmax_v[pl.ds((s * _CPS + i) * _L, _L)] = acc
            return jnp.maximum(sacc, acc), jnp.minimum(gmin, accmin)

        sacc, gminv = lax.fori_loop(
            0, _CPS, _cbody,
            (jnp.full((_L,), -jnp.inf, jnp.float32), gminv),
        )
        smax_v[pl.ds(s * _L, _L)] = sacc

    m = jnp.min(gminv)
    m_v = jnp.full((_L,), m, jnp.float32)

    picks = []
    for t in range(_TOPK):
        macc = smax_v[pl.ds(0, _L)]
        for s in range(1, _NSUP):
            macc = jnp.maximum(macc, smax_v[pl.ds(s * _L, _L)])
        big_m = jnp.max(macc)

        sidxv = jnp.full((_L,), _BIG, jnp.int32)
        for s in range(_NSUP):
            sm = smax_v[pl.ds(s * _L, _L)]
            sidxv = jnp.minimum(sidxv, jnp.where(sm == big_m, s, _BIG))
        sidx = jnp.min(sidxv)

        cbase = sidx * _CPS
        cidxv = jnp.full((_L,), _BIG, jnp.int32)
        for k in range(_CPS):
            cm = cmax_v[pl.ds((cbase + k) * _L, _L)]
            cidxv = jnp.minimum(cidxv, jnp.where(cm == big_m, cbase + k, _BIG))
        cidx = jnp.min(cidxv)

        base = cidx * _C
        idxacc = jnp.full((_L,), _BIG, jnp.int32)
        for i in range(_NVPC):
            v = row_v[pl.ds(base + i * _L, _L)]
            idxacc = jnp.minimum(
                idxacc, jnp.where(v == big_m, base + i * _L + iota, _BIG)
            )
        gidx = jnp.min(idxacc)
        picks.append(gidx)

        widx = gidx - _RADIUS + iota
        wmask = (iota < 2 * _RADIUS) & (gidx >= _RADIUS) & (widx < _N)
        widx_c = jnp.clip(widx, 0, _N - 1)
        plsc.store_scatter(row_v, [widx_c], m_v, mask=wmask)

        if t < _TOPK - 1:
            ws = jnp.maximum(gidx - _RADIUS, 0)
            we = jnp.minimum(gidx + _RADIUS, _N) - 1
            ca = ws // _C
            cb = we // _C
            _rescan_chunk(row_v, cmax_v, ca)
            _rescan_chunk(row_v, cmax_v, cb)
            _rebuild_super(cmax_v, smax_v, ca // _CPS)
            _rebuild_super(cmax_v, smax_v, cb // _CPS)

    return picks


def _pick_vec_mask(picks, iota):
    pv = jnp.full((_L,), picks[0], jnp.int32)
    vmask = iota == 0
    for t in range(1, _TOPK):
        pv = jnp.where(iota == t, picks[t], pv)
        dup = picks[t] == picks[0]
        for s in range(1, t):
            dup = dup | (picks[t] == picks[s])
        vmask = vmask | ((iota == t) & jnp.logical_not(dup))
    return pv, vmask


@functools.partial(
    pl.kernel,
    out_type=jax.ShapeDtypeStruct((_ROWS, _N), jnp.float32),
    mesh=_mesh,
    compiler_params=pltpu.CompilerParams(needs_layout_passes=False),
    scratch_types=[
        pltpu.VMEM((_N,), jnp.float32),
        pltpu.VMEM((_N,), jnp.float32),
        pltpu.VMEM((_N,), jnp.float32),
        pltpu.VMEM((_NCHUNK * _L,), jnp.float32),
        pltpu.VMEM((_NSUP * _L,), jnp.float32),
        pltpu.SemaphoreType.DMA,
        pltpu.SemaphoreType.DMA,
        pltpu.SemaphoreType.DMA,
    ],
)
def _wta_sc(x_hbm, out_hbm, rowa_v, rowb_v, out_v, cmax_v, smax_v,
            sem_a, sem_b, sem_o):
    wid = lax.axis_index("s")
    rows = [wid * _RPT + k for k in range(_RPT)]
    iota = lax.iota(jnp.int32, _L)
    zero_v = jnp.zeros((_L,), jnp.float32)
    one_v = jnp.ones((_L,), jnp.float32)
    bufs = [rowa_v, rowb_v]
    sems = [sem_a, sem_b]

    dma = {
        0: pltpu.async_copy(x_hbm.at[rows[0]], bufs[0], sems[0]),
        1: pltpu.async_copy(x_hbm.at[rows[1]], bufs[1], sems[1]),
    }

    def _zbody(i, carry):
        for u in range(16):
            out_v[pl.ds(i * 256 + u * _L, _L)] = zero_v
        return carry

    lax.fori_loop(0, _N // 256, _zbody, jnp.int32(0))

    out_dma = None
    prev = None
    for k in range(_RPT):
        buf = bufs[k % 2]
        picks = _compute_picks(buf, cmax_v, smax_v, iota, dma[k])
        if k + 2 < _RPT:
            dma[k + 2] = pltpu.async_copy(
                x_hbm.at[rows[k + 2]], buf, sems[k % 2]
            )
        pv, mask = _pick_vec_mask(picks, iota)
        if out_dma is not None:
            out_dma.wait()
            plsc.store_scatter(out_v, [prev[0]], zero_v, mask=prev[1])
        plsc.store_scatter(out_v, [pv], one_v, mask=mask)
        if k < _RPT - 1:
            out_dma = pltpu.async_copy(out_v, out_hbm.at[rows[k]], sem_o)
            prev = (pv, mask)
        else:
            pltpu.sync_copy(out_v, out_hbm.at[rows[k]])


def kernel(x):
    return _wta_sc(x)


# speedup vs baseline: 1.2005x; 1.2005x over previous
import functools

import jax
import jax.numpy as jnp
from jax import lax
from jax.experimental import pallas as pl
from jax.experimental.pallas import tpu as pltpu
from jax.experimental.pallas import tpu_sc as plsc

_TOPK = 5
_RADIUS = 5
_ROWS = 64
_N = 32768
_L = 16
_C = 256
_NCHUNK = _N // _C
_NVPC = _C // _L
_NSUP = 8
_CPS = _NCHUNK // _NSUP
_SEC = _N // _NSUP
_BIG = 1 << 30

_mesh = plsc.VectorSubcoreMesh(
    core_axis_name="c", subcore_axis_name="s", num_cores=2, num_subcores=16
)


def _rescan_chunk(row_v, cmax_v, c):
    base = c * _C
    acc = row_v[pl.ds(base, _L)]
    for i in range(1, _NVPC):
        acc = jnp.maximum(acc, row_v[pl.ds(base + i * _L, _L)])
    cmax_v[pl.ds(c * _L, _L)] = acc


def _rebuild_super(cmax_v, smax_v, s):
    base = s * _CPS * _L
    acc = cmax_v[pl.ds(base, _L)]
    for k in range(1, _CPS):
        acc = jnp.maximum(acc, cmax_v[pl.ds(base + k * _L, _L)])
    smax_v[pl.ds(s * _L, _L)] = acc


def _compute_picks(row_v, cmax_v, smax_v, iota, sec_dmas):
    sec_dmas.wait()
    gminv = jnp.full((_L,), jnp.inf, jnp.float32)
    for s in range(_NSUP):

        @plsc.parallel_loop(
            0, _CPS, 1, unroll=2,
            carry=(jnp.full((_L,), -jnp.inf, jnp.float32), gminv),
        )
        def _p1(i, carry, s=s):
            sacc, gmin = carry
            base = (s * _CPS + i) * _C
            acc = row_v[pl.ds(base, _L)]
            accmin = acc
            for k in range(1, _NVPC):
                v = row_v[pl.ds(base + k * _L, _L)]
                acc = jnp.maximum(acc, v)
                accmin = jnp.minimum(accmin, v)
            cmax_v[pl.ds((s * _CPS + i) * _L, _L)] = acc
            return jnp.maximum(sacc, acc), jnp.minimum(gmin, accmin)

        sacc, gminv = _p1
        smax_v[pl.ds(s * _L, _L)] = sacc

    m = jnp.min(gminv)
    m_v = jnp.full((_L,), m, jnp.float32)

    picks = []
    for t in range(_TOPK):
        macc = smax_v[pl.ds(0, _L)]
        for s in range(1, _NSUP):
            macc = jnp.maximum(macc, smax_v[pl.ds(s * _L, _L)])
        big_m = jnp.max(macc)

        sidxv = jnp.full((_L,), _BIG, jnp.int32)
        for s in range(_NSUP):
            sm = smax_v[pl.ds(s * _L, _L)]
            sidxv = jnp.minimum(sidxv, jnp.where(sm == big_m, s, _BIG))
        sidx = jnp.min(sidxv)

        cbase = sidx * _CPS
        cidxv = jnp.full((_L,), _BIG, jnp.int32)
        for k in range(_CPS):
            cm = cmax_v[pl.ds((cbase + k) * _L, _L)]
            cidxv = jnp.minimum(cidxv, jnp.where(cm == big_m, cbase + k, _BIG))
        cidx = jnp.min(cidxv)

        base = cidx * _C
        idxacc = jnp.full((_L,), _BIG, jnp.int32)
        for i in range(_NVPC):
            v = row_v[pl.ds(base + i * _L, _L)]
            idxacc = jnp.minimum(
                idxacc, jnp.where(v == big_m, base + i * _L + iota, _BIG)
            )
        gidx = jnp.min(idxacc)
        picks.append(gidx)

        widx = gidx - _RADIUS + iota
        wmask = (iota < 2 * _RADIUS) & (gidx >= _RADIUS) & (widx < _N)
        widx_c = jnp.clip(widx, 0, _N - 1)
        plsc.store_scatter(row_v, [widx_c], m_v, mask=wmask)

        if t < _TOPK - 1:
            ws = jnp.maximum(gidx - _RADIUS, 0)
            we = jnp.minimum(gidx + _RADIUS, _N) - 1
            ca = ws // _C
            cb = we // _C
            _rescan_chunk(row_v, cmax_v, ca)
            _rescan_chunk(row_v, cmax_v, cb)
            _rebuild_super(cmax_v, smax_v, ca // _CPS)
            _rebuild_super(cmax_v, smax_v, cb // _CPS)

    return picks


def _pick_vec_mask(picks, iota):
    pv = jnp.full((_L,), picks[0], jnp.int32)
    vmask = iota == 0
    for t in range(1, _TOPK):
        pv = jnp.where(iota == t, picks[t], pv)
        dup = picks[t] == picks[0]
        for s in range(1, t):
            dup = dup | (picks[t] == picks[s])
        vmask = vmask | ((iota == t) & jnp.logical_not(dup))
    return pv, vmask


@functools.partial(
    pl.kernel,
    out_type=jax.ShapeDtypeStruct((_ROWS, _N), jnp.float32),
    mesh=_mesh,
    compiler_params=pltpu.CompilerParams(needs_layout_passes=False),
    scratch_types=[
        pltpu.VMEM((_N,), jnp.float32),
        pltpu.VMEM((_N,), jnp.float32),
        pltpu.VMEM((_N,), jnp.float32),
        pltpu.VMEM((_NCHUNK * _L,), jnp.float32),
        pltpu.VMEM((_NSUP * _L,), jnp.float32),
        pltpu.SemaphoreType.DMA,
        pltpu.SemaphoreType.DMA,
        pltpu.SemaphoreType.DMA,
    ],
)
def _wta_sc(x_hbm, out_hbm, rowa_v, rowb_v, out_v, cmax_v, smax_v,
            sem_a, sem_b, sem_o):
    wid = lax.axis_index("s") * 2 + lax.axis_index("c")
    ra = wid * 2
    rb = ra + 1
    iota = lax.iota(jnp.int32, _L)
    zero_v = jnp.zeros((_L,), jnp.float32)
    one_v = jnp.ones((_L,), jnp.float32)

    dma_a = pltpu.async_copy(x_hbm.at[ra], rowa_v, sem_a)
    dma_b = pltpu.async_copy(x_hbm.at[rb], rowb_v, sem_b)

    def _zbody(i, carry):
        for u in range(16):
            out_v[pl.ds(i * 256 + u * _L, _L)] = zero_v
        return carry

    lax.fori_loop(0, _N // 256, _zbody, jnp.int32(0))

    picks_a = _compute_picks(rowa_v, cmax_v, smax_v, iota, dma_a)
    pva, maska = _pick_vec_mask(picks_a, iota)
    plsc.store_scatter(out_v, [pva], one_v, mask=maska)
    out_a = pltpu.async_copy(out_v, out_hbm.at[ra], sem_o)

    picks_b = _compute_picks(rowb_v, cmax_v, smax_v, iota, dma_b)
    out_a.wait()
    plsc.store_scatter(out_v, [pva], zero_v, mask=maska)
    pvb, maskb = _pick_vec_mask(picks_b, iota)
    plsc.store_scatter(out_v, [pvb], one_v, mask=maskb)
    pltpu.sync_copy(out_v, out_hbm.at[rb])


def kernel(x):
    return _wta_sc(x)


# speedup vs baseline: 1.2602x; 1.0497x over previous
import functools

import jax
import jax.numpy as jnp
from jax import lax
from jax.experimental import pallas as pl
from jax.experimental.pallas import tpu as pltpu
from jax.experimental.pallas import tpu_sc as plsc

_TOPK = 5
_RADIUS = 5
_ROWS = 64
_N = 32768
_L = 16
_C = 256
_NCHUNK = _N // _C
_NVPC = _C // _L
_NSUP = 8
_CPS = _NCHUNK // _NSUP
_SEC = _N // _NSUP
_BIG = 1 << 30

_mesh = plsc.VectorSubcoreMesh(
    core_axis_name="c", subcore_axis_name="s", num_cores=2, num_subcores=16
)


def _rescan_chunk(row_v, cmax_v, c):
    base = c * _C
    acc = row_v[pl.ds(base, _L)]
    for i in range(1, _NVPC):
        acc = jnp.maximum(acc, row_v[pl.ds(base + i * _L, _L)])
    cmax_v[pl.ds(c * _L, _L)] = acc


def _rebuild_super(cmax_v, smax_v, s):
    base = s * _CPS * _L
    acc = cmax_v[pl.ds(base, _L)]
    for k in range(1, _CPS):
        acc = jnp.maximum(acc, cmax_v[pl.ds(base + k * _L, _L)])
    smax_v[pl.ds(s * _L, _L)] = acc


def _compute_picks(row_v, cmax_v, smax_v, iota, sec_dmas):
    gminv = jnp.full((_L,), jnp.inf, jnp.float32)
    waits = dict((fs, d) for d, fs in sec_dmas)
    for s in range(_NSUP):
        if s in waits:
            waits[s].wait()

        def _cbody(i, carry, s=s):
            sacc, gmin = carry
            base = (s * _CPS + i) * _C
            acc = row_v[pl.ds(base, _L)]
            accmin = acc
            for k in range(1, _NVPC):
                v = row_v[pl.ds(base + k * _L, _L)]
                acc = jnp.maximum(acc, v)
                accmin = jnp.minimum(accmin, v)
            cmax_v[pl.ds((s * _CPS + i) * _L, _L)] = acc
            return jnp.maximum(sacc, acc), jnp.minimum(gmin, accmin)

        sacc, gminv = lax.fori_loop(
            0, _CPS, _cbody,
            (jnp.full((_L,), -jnp.inf, jnp.float32), gminv),
        )
        smax_v[pl.ds(s * _L, _L)] = sacc

    m = jnp.min(gminv)
    m_v = jnp.full((_L,), m, jnp.float32)

    picks = []
    for t in range(_TOPK):
        macc = smax_v[pl.ds(0, _L)]
        for s in range(1, _NSUP):
            macc = jnp.maximum(macc, smax_v[pl.ds(s * _L, _L)])
        big_m = jnp.max(macc)

        sidxv = jnp.full((_L,), _BIG, jnp.int32)
        for s in range(_NSUP):
            sm = smax_v[pl.ds(s * _L, _L)]
            sidxv = jnp.minimum(sidxv, jnp.where(sm == big_m, s, _BIG))
        sidx = jnp.min(sidxv)

        cbase = sidx * _CPS
        cidxv = jnp.full((_L,), _BIG, jnp.int32)
        for k in range(_CPS):
            cm = cmax_v[pl.ds((cbase + k) * _L, _L)]
            cidxv = jnp.minimum(cidxv, jnp.where(cm == big_m, cbase + k, _BIG))
        cidx = jnp.min(cidxv)

        base = cidx * _C
        idxacc = jnp.full((_L,), _BIG, jnp.int32)
        for i in range(_NVPC):
            v = row_v[pl.ds(base + i * _L, _L)]
            idxacc = jnp.minimum(
                idxacc, jnp.where(v == big_m, base + i * _L + iota, _BIG)
            )
        gidx = jnp.min(idxacc)
        picks.append(gidx)

        widx = gidx - _RADIUS + iota
        wmask = (iota < 2 * _RADIUS) & (gidx >= _RADIUS) & (widx < _N)
        widx_c = jnp.clip(widx, 0, _N - 1)
        plsc.store_scatter(row_v, [widx_c], m_v, mask=wmask)

        if t < _TOPK - 1:
            ws = jnp.maximum(gidx - _RADIUS, 0)
            we = jnp.minimum(gidx + _RADIUS, _N) - 1
            ca = ws // _C
            cb = we // _C
            _rescan_chunk(row_v, cmax_v, ca)
            _rescan_chunk(row_v, cmax_v, cb)
            _rebuild_super(cmax_v, smax_v, ca // _CPS)
            _rebuild_super(cmax_v, smax_v, cb // _CPS)

    return picks


def _pick_vec_mask(picks, iota):
    pv = jnp.full((_L,), picks[0], jnp.int32)
    vmask = iota == 0
    for t in range(1, _TOPK):
        pv = jnp.where(iota == t, picks[t], pv)
        dup = picks[t] == picks[0]
        for s in range(1, t):
            dup = dup | (picks[t] == picks[s])
        vmask = vmask | ((iota == t) & jnp.logical_not(dup))
    return pv, vmask


@functools.partial(
    pl.kernel,
    out_type=jax.ShapeDtypeStruct((_ROWS, _N), jnp.float32),
    mesh=_mesh,
    compiler_params=pltpu.CompilerParams(needs_layout_passes=False),
    scratch_types=[
        pltpu.VMEM((_N,), jnp.float32),
        pltpu.VMEM((_N,), jnp.float32),
        pltpu.VMEM((_N,), jnp.float32),
        pltpu.VMEM((_NCHUNK * _L,), jnp.float32),
        pltpu.VMEM((_NSUP * _L,), jnp.float32),
        pltpu.SemaphoreType.DMA,
        pltpu.SemaphoreType.DMA,
        pltpu.SemaphoreType.DMA,
        pltpu.SemaphoreType.DMA,
    ],
)
def _wta_sc(x_hbm, out_hbm, rowa_v, rowb_v, out_v, cmax_v, smax_v,
            sem_a1, sem_a2, sem_b, sem_o):
    wid = lax.axis_index("s") * 2 + lax.axis_index("c")
    ra = wid * 2
    rb = ra + 1
    iota = lax.iota(jnp.int32, _L)
    zero_v = jnp.zeros((_L,), jnp.float32)
    one_v = jnp.ones((_L,), jnp.float32)

    half = _N // 2
    dma_a1 = pltpu.async_copy(
        x_hbm.at[ra, pl.ds(0, half)], rowa_v.at[pl.ds(0, half)], sem_a1
    )
    dma_a2 = pltpu.async_copy(
        x_hbm.at[ra, pl.ds(half, half)], rowa_v.at[pl.ds(half, half)], sem_a2
    )
    dma_b = pltpu.async_copy(x_hbm.at[rb], rowb_v, sem_b)

    def _zbody(i, carry):
        for u in range(16):
            out_v[pl.ds(i * 256 + u * _L, _L)] = zero_v
        return carry

    lax.fori_loop(0, _N // 256, _zbody, jnp.int32(0))

    picks_a = _compute_picks(
        rowa_v, cmax_v, smax_v, iota,
        [(dma_a1, 0), (dma_a2, _NSUP // 2)],
    )
    pva, maska = _pick_vec_mask(picks_a, iota)
    plsc.store_scatter(out_v, [pva], one_v, mask=maska)
    out_a = pltpu.async_copy(out_v, out_hbm.at[ra], sem_o)

    picks_b = _compute_picks(rowb_v, cmax_v, smax_v, iota, [(dma_b, 0)])
    out_a.wait()
    plsc.store_scatter(out_v, [pva], zero_v, mask=maska)
    pvb, maskb = _pick_vec_mask(picks_b, iota)
    plsc.store_scatter(out_v, [pvb], one_v, mask=maskb)
    pltpu.sync_copy(out_v, out_hbm.at[rb])


def kernel(x):
    return _wta_sc(x)
